# baseline (speedup 1.0000x reference)
import functools

import jax
import jax.numpy as jnp
from jax import lax
from jax.experimental import pallas as pl
from jax.experimental.pallas import tpu as pltpu
from jax.experimental.pallas import tpu_sc as plsc

T = 16384
D = 768
E = 64
CAP = 256
S = E * CAP

BT = 1024
NB = T // BT

NW = 32
TPW = T // NW
G = 64
NCH = TPW // G
IDXW = 128
NSC = TPW // IDXW


def _router_body(x_ref, wg_ref, scat_ref, flat_ref, gate_ref, cnt_ref,
                 laux_ref, csum_ref, gsum_ref):
    b = pl.program_id(0)

    @pl.when(b == 0)
    def _init():
        csum_ref[...] = jnp.zeros_like(csum_ref)
        gsum_ref[...] = jnp.zeros_like(gsum_ref)

    logits = jnp.dot(x_ref[...], wg_ref[...],
                     preferred_element_type=jnp.float32)
    m = jnp.max(logits, axis=1, keepdims=True)
    p = jnp.exp(logits - m)
    gates = p / jnp.sum(p, axis=1, keepdims=True)

    gmax = jnp.max(gates, axis=1, keepdims=True)
    eidx = lax.broadcasted_iota(jnp.int32, (BT, E), 1)
    idx1 = jnp.min(jnp.where(gates >= gmax, eidx, E),
                   axis=1, keepdims=True)
    onehot = (eidx == idx1).astype(jnp.float32)

    cs = onehot
    k = 1
    while k < BT:
        cs = cs + jnp.pad(cs, ((k, 0), (0, 0)))[:BT]
        k *= 2

    base = csum_ref[...]
    loc = cs - 1.0 + base
    loc_tok = jnp.sum(loc * onehot, axis=1, keepdims=True)
    keep = loc_tok < float(CAP)
    loc_i = loc_tok.astype(jnp.int32)
    flat = idx1 * CAP + loc_i
    tok_id = lax.broadcasted_iota(jnp.int32, (BT, 1), 0) + b * BT

    scat_ref[...] = jnp.where(keep, flat, S + tok_id)
    flat_ref[...] = jnp.where(keep, flat, S)
    gate_ref[...] = jnp.where(keep, gmax, 0.0)

    csum_ref[...] = base + jnp.sum(onehot, axis=0, keepdims=True)
    gsum_ref[...] = gsum_ref[...] + jnp.sum(gates, axis=0, keepdims=True)

    @pl.when(b == NB - 1)
    def _fin():
        cnt = csum_ref[...]
        cnt_ref[...] = (cnt + 0.5).astype(jnp.int32)
        me = gsum_ref[...] * (1.0 / T)
        ce = cnt * (1.0 / T)
        laux_ref[...] = jnp.sum(me * ce, keepdims=True) * float(E)


def _router(x, wg):
    return pl.pallas_call(
        _router_body,
        grid=(NB,),
        in_specs=[
            pl.BlockSpec((BT, D), lambda i: (i, 0)),
            pl.BlockSpec((D, E), lambda i: (0, 0)),
        ],
        out_specs=[
            pl.BlockSpec((BT, 1), lambda i: (i, 0)),
            pl.BlockSpec((BT, 1), lambda i: (i, 0)),
            pl.BlockSpec((BT, 1), lambda i: (i, 0)),
            pl.BlockSpec((1, E), lambda i: (0, 0)),
            pl.BlockSpec((1, 1), lambda i: (0, 0)),
        ],
        out_shape=[
            jax.ShapeDtypeStruct((T, 1), jnp.int32),
            jax.ShapeDtypeStruct((T, 1), jnp.int32),
            jax.ShapeDtypeStruct((T, 1), jnp.float32),
            jax.ShapeDtypeStruct((1, E), jnp.int32),
            jax.ShapeDtypeStruct((1, 1), jnp.float32),
        ],
        scratch_shapes=[
            pltpu.VMEM((1, E), jnp.float32),
            pltpu.VMEM((1, E), jnp.float32),
        ],
    )(x, wg)


def _scatter_body(scat_hbm, gate_hbm, map_hbm, sgate_hbm, idx_v, ids_v, g_v, sem):
    c = lax.axis_index("c")
    s = lax.axis_index("s")
    wid = s * 2 + c
    base = wid * TPW
    pltpu.sync_copy(scat_hbm.at[wid], idx_v)
    pltpu.sync_copy(gate_hbm.at[wid], g_v)
    for j in range(NSC):
        for i in range(IDXW // 16):
            ids_v[j, pl.ds(i * 16, 16)] = (
                base + j * IDXW + i * 16 + lax.iota(jnp.int32, 16))
    copies = []
    for j in range(NSC):
        copies.append(pltpu.async_copy(ids_v.at[j], map_hbm.at[idx_v.at[j]], sem))
        copies.append(pltpu.async_copy(g_v.at[j], sgate_hbm.at[idx_v.at[j]], sem))
    for cp in copies:
        cp.wait()


def _dispatch_body(map_hbm, x_hbm, disp_hbm,
                   gidx_v, gidx2_v, buf0, buf1,
                   gsem0, gsem1, osem0, osem1):
    c = lax.axis_index("c")
    s = lax.axis_index("s")
    wid = s * 2 + c
    sbase = wid * TPW
    pltpu.sync_copy(map_hbm.at[pl.ds(sbase, TPW)], gidx_v)

    for j in range(NCH):
        for i in range(G // 16):
            v = gidx_v[pl.ds(j * G + i * 16, 16)]
            gidx2_v[j, pl.ds(i * 16, 16)] = lax.bitwise_and(v, T - 1)

    bufs = (buf0, buf1)
    gsems = (gsem0, gsem1)
    osems = (osem0, osem1)
    outcp = [None, None]
    incp = pltpu.async_copy(x_hbm.at[gidx2_v.at[0]], buf0, gsem0)
    for j in range(NCH):
        b = j & 1
        nb = 1 - b
        incp.wait()
        if j + 1 < NCH:
            if outcp[nb] is not None:
                outcp[nb].wait()
            incp = pltpu.async_copy(
                x_hbm.at[gidx2_v.at[j + 1]], bufs[nb], gsems[nb])
        outcp[b] = pltpu.async_copy(
            bufs[b], disp_hbm.at[pl.ds(sbase + j * G, G)], osems[b])
    if outcp[(NCH - 2) & 1] is not None:
        outcp[(NCH - 2) & 1].wait()
    outcp[(NCH - 1) & 1].wait()


def _expert_body(disp_ref, sg_ref, we_ref, be_ref, out_ref):
    e = pl.program_id(0)

    @pl.when(e == E)
    def _zero():
        out_ref[...] = jnp.zeros_like(out_ref)

    @pl.when(e < E)
    def _ffn():
        xb = disp_ref[0]
        sg = sg_ref[0]
        acc = jnp.dot(xb * sg, we_ref[0], preferred_element_type=jnp.float32)
        out_ref[...] = acc + sg * be_ref[0]


def _experts(disp, sgate, We, be):
    return pl.pallas_call(
        _expert_body,
        grid=(E + 1,),
        in_specs=[
            pl.BlockSpec((1, CAP, D), lambda i: (jnp.minimum(i, E - 1), 0, 0)),
            pl.BlockSpec((1, CAP, 1), lambda i: (jnp.minimum(i, E - 1), 0, 0)),
            pl.BlockSpec((1, D, D), lambda i: (jnp.minimum(i, E - 1), 0, 0)),
            pl.BlockSpec((1, 1, D), lambda i: (jnp.minimum(i, E - 1), 0, 0)),
        ],
        out_specs=pl.BlockSpec((CAP, D), lambda i: (i, 0)),
        out_shape=jax.ShapeDtypeStruct(((E + 1) * CAP, D), jnp.float32),
    )(disp, sgate, We, be)


def _combine_body(flat_hbm, eo_hbm, out_hbm, idx_v, buf0, buf1,
                  gsem0, gsem1, osem0, osem1):
    c = lax.axis_index("c")
    s = lax.axis_index("s")
    wid = s * 2 + c
    base = wid * TPW
    pltpu.sync_copy(flat_hbm.at[wid], idx_v)
    bufs = (buf0, buf1)
    gsems = (gsem0, gsem1)
    osems = (osem0, osem1)
    outcp = [None, None]
    incp = pltpu.async_copy(eo_hbm.at[idx_v.at[0]], buf0, gsem0)
    for j in range(NCH):
        b = j & 1
        nb = 1 - b
        incp.wait()
        if j + 1 < NCH:
            if outcp[nb] is not None:
                outcp[nb].wait()
            incp = pltpu.async_copy(
                eo_hbm.at[idx_v.at[j + 1]], bufs[nb], gsems[nb])
        outcp[b] = pltpu.async_copy(
            bufs[b], out_hbm.at[pl.ds(base + j * G, G)], osems[b])
    if outcp[(NCH - 2) & 1] is not None:
        outcp[(NCH - 2) & 1].wait()
    outcp[(NCH - 1) & 1].wait()


@functools.lru_cache(maxsize=1)
def _sc_kernels():
    mesh = plsc.VectorSubcoreMesh(core_axis_name="c", subcore_axis_name="s",
                                  num_cores=2)
    scatter_k = pl.kernel(
        _scatter_body,
        out_type=(jax.ShapeDtypeStruct((S + T,), jnp.int32),
                  jax.ShapeDtypeStruct((S + T,), jnp.float32)),
        mesh=mesh,
        scratch_types=[
            pltpu.VMEM((NSC, IDXW), jnp.int32),
            pltpu.VMEM((NSC, IDXW), jnp.int32),
            pltpu.VMEM((NSC, IDXW), jnp.float32),
            pltpu.SemaphoreType.DMA,
        ],
    )
    dispatch_k = pl.kernel(
        _dispatch_body,
        out_type=jax.ShapeDtypeStruct((S, D), jnp.float32),
        mesh=mesh,
        scratch_types=[
            pltpu.VMEM((TPW,), jnp.int32),
            pltpu.VMEM((NCH, G), jnp.int32),
            pltpu.VMEM((G, D), jnp.float32),
            pltpu.VMEM((G, D), jnp.float32),
            pltpu.SemaphoreType.DMA,
            pltpu.SemaphoreType.DMA,
            pltpu.SemaphoreType.DMA,
            pltpu.SemaphoreType.DMA,
        ],
    )
    combine_k = pl.kernel(
        _combine_body,
        out_type=jax.ShapeDtypeStruct((T, D), jnp.float32),
        mesh=mesh,
        scratch_types=[
            pltpu.VMEM((NCH, G), jnp.int32),
            pltpu.VMEM((G, D), jnp.float32),
            pltpu.VMEM((G, D), jnp.float32),
            pltpu.SemaphoreType.DMA,
            pltpu.SemaphoreType.DMA,
            pltpu.SemaphoreType.DMA,
            pltpu.SemaphoreType.DMA,
        ],
    )
    return scatter_k, dispatch_k, combine_k


def kernel(hidden_states, wg, We, be):
    B, SEQ, _ = hidden_states.shape
    x = hidden_states.reshape(T, D)
    scatter_k, dispatch_k, combine_k = _sc_kernels()

    scat, flat, gate, cnt, laux = _router(x, wg)

    scat_r = scat.reshape(NW, NSC, IDXW)
    gate_r = gate.reshape(NW, NSC, IDXW)
    slot_map, slot_gate = scatter_k(scat_r, gate_r)
    disp = dispatch_k(slot_map, x)

    disp3 = disp.reshape(E, CAP, D)
    sg3 = slot_gate[:S].reshape(E, CAP, 1)
    eo = _experts(disp3, sg3, We, be.reshape(E, 1, D))

    flat_r = flat.reshape(NW, NCH, G)
    out = combine_k(flat_r, eo)

    return (out.reshape(B, SEQ, D), laux[0, 0], cnt.reshape(E))


# speedup vs baseline: 1.3601x; 1.3601x over previous
import functools

import jax
import jax.numpy as jnp
from jax import lax
from jax.experimental import pallas as pl
from jax.experimental.pallas import tpu as pltpu
from jax.experimental.pallas import tpu_sc as plsc

T = 16384
D = 768
E = 64
CAP = 256
S = E * CAP

BT = 1024
NB = T // BT

NW = 32
TPW = T // NW
G = 64
NCH = TPW // G
IDXW = 128
NSC = TPW // IDXW


def _router_body(x_ref, wg_ref, scat_ref, flat_ref, gate_ref, cnt_ref,
                 laux_ref, csum_ref, gsum_ref):
    b = pl.program_id(0)

    @pl.when(b == 0)
    def _init():
        csum_ref[...] = jnp.zeros_like(csum_ref)
        gsum_ref[...] = jnp.zeros_like(gsum_ref)

    logits = jnp.dot(x_ref[...], wg_ref[...],
                     preferred_element_type=jnp.float32)
    m = jnp.max(logits, axis=1, keepdims=True)
    p = jnp.exp(logits - m)
    gates = p / jnp.sum(p, axis=1, keepdims=True)

    gmax = jnp.max(gates, axis=1, keepdims=True)
    eidx = lax.broadcasted_iota(jnp.int32, (BT, E), 1)
    idx1 = jnp.min(jnp.where(gates >= gmax, eidx, E),
                   axis=1, keepdims=True)
    onehot = (eidx == idx1).astype(jnp.float32)

    cs = onehot
    k = 1
    while k < BT:
        cs = cs + jnp.pad(cs, ((k, 0), (0, 0)))[:BT]
        k *= 2

    base = csum_ref[...]
    loc = cs - 1.0 + base
    loc_tok = jnp.sum(loc * onehot, axis=1, keepdims=True)
    keep = loc_tok < float(CAP)
    loc_i = loc_tok.astype(jnp.int32)
    flat = idx1 * CAP + loc_i
    tok_id = lax.broadcasted_iota(jnp.int32, (BT, 1), 0) + b * BT

    scat_ref[...] = jnp.where(keep, flat, S + tok_id)
    flat_ref[...] = jnp.where(keep, flat, S)
    gate_ref[...] = jnp.where(keep, gmax, 0.0)

    csum_ref[...] = base + jnp.sum(onehot, axis=0, keepdims=True)
    gsum_ref[...] = gsum_ref[...] + jnp.sum(gates, axis=0, keepdims=True)

    @pl.when(b == NB - 1)
    def _fin():
        cnt = csum_ref[...]
        cnt_ref[...] = (cnt + 0.5).astype(jnp.int32)
        me = gsum_ref[...] * (1.0 / T)
        ce = cnt * (1.0 / T)
        laux_ref[...] = jnp.sum(me * ce, keepdims=True) * float(E)


def _router(x, wg):
    return pl.pallas_call(
        _router_body,
        grid=(NB,),
        in_specs=[
            pl.BlockSpec((BT, D), lambda i: (i, 0)),
            pl.BlockSpec((D, E), lambda i: (0, 0)),
        ],
        out_specs=[
            pl.BlockSpec((BT, 1), lambda i: (i, 0)),
            pl.BlockSpec((BT, 1), lambda i: (i, 0)),
            pl.BlockSpec((BT, 1), lambda i: (i, 0)),
            pl.BlockSpec((1, E), lambda i: (0, 0)),
            pl.BlockSpec((1, 1), lambda i: (0, 0)),
        ],
        out_shape=[
            jax.ShapeDtypeStruct((T, 1), jnp.int32),
            jax.ShapeDtypeStruct((T, 1), jnp.int32),
            jax.ShapeDtypeStruct((T, 1), jnp.float32),
            jax.ShapeDtypeStruct((1, E), jnp.int32),
            jax.ShapeDtypeStruct((1, 1), jnp.float32),
        ],
        scratch_shapes=[
            pltpu.VMEM((1, E), jnp.float32),
            pltpu.VMEM((1, E), jnp.float32),
        ],
    )(x, wg)


TPT = T // 16
NSC2 = TPT // IDXW
HS = S // 2


def _dispatch_body(scat_hbm, gate_hbm, x_hbm, sgate_hbm, disp_hbm,
                   map_sh, gate_sh,
                   idx_v, ids_v, g_v, gidx_v, gidx2_v, gstage_v, buf0, buf1,
                   ssem, gsem0, gsem1, osem0, osem1):
    c = lax.axis_index("c")
    s = lax.axis_index("s")

    tbase = s * TPT
    pltpu.sync_copy(scat_hbm.at[s], idx_v)
    pltpu.sync_copy(gate_hbm.at[s], g_v)
    lo = c * HS
    for j in range(NSC2):
        for i in range(IDXW // 16):
            off = pl.ds(i * 16, 16)
            tok = tbase + j * IDXW + i * 16 + lax.iota(jnp.int32, 16)
            v = idx_v[j, off]
            inhalf = jnp.logical_and(v >= lo, v < lo + HS)
            idx_v[j, off] = jnp.where(inhalf, v - lo, HS + tok)
            ids_v[j, off] = tok
    copies = []
    for j in range(NSC2):
        copies.append(pltpu.async_copy(ids_v.at[j], map_sh.at[idx_v.at[j]], ssem))
        copies.append(pltpu.async_copy(g_v.at[j], gate_sh.at[idx_v.at[j]], ssem))
    for cp in copies:
        cp.wait()
    plsc.subcore_barrier()

    lslot = s * TPW
    sbase = c * HS + lslot
    pltpu.sync_copy(map_sh.at[pl.ds(lslot, TPW)], gidx_v)
    pltpu.sync_copy(gate_sh.at[pl.ds(lslot, TPW)], gstage_v)
    pltpu.sync_copy(gstage_v, sgate_hbm.at[pl.ds(sbase, TPW)])

    for j in range(NCH):
        for i in range(G // 16):
            v = gidx_v[pl.ds(j * G + i * 16, 16)]
            gidx2_v[j, pl.ds(i * 16, 16)] = lax.bitwise_and(v, T - 1)

    bufs = (buf0, buf1)
    gsems = (gsem0, gsem1)
    osems = (osem0, osem1)
    outcp = [None, None]
    incp = pltpu.async_copy(x_hbm.at[gidx2_v.at[0]], buf0, gsem0)
    for j in range(NCH):
        b = j & 1
        nb = 1 - b
        incp.wait()
        if j + 1 < NCH:
            if outcp[nb] is not None:
                outcp[nb].wait()
            incp = pltpu.async_copy(
                x_hbm.at[gidx2_v.at[j + 1]], bufs[nb], gsems[nb])
        outcp[b] = pltpu.async_copy(
            bufs[b], disp_hbm.at[pl.ds(sbase + j * G, G)], osems[b])
    if outcp[(NCH - 2) & 1] is not None:
        outcp[(NCH - 2) & 1].wait()
    outcp[(NCH - 1) & 1].wait()


def _expert_body(disp_ref, sg_ref, we_ref, be_ref, out_ref):
    e = pl.program_id(0)

    @pl.when(e == E)
    def _zero():
        out_ref[...] = jnp.zeros_like(out_ref)

    @pl.when(e < E)
    def _ffn():
        xb = disp_ref[0]
        sg = sg_ref[0]
        acc = jnp.dot(xb * sg, we_ref[0], preferred_element_type=jnp.float32)
        out_ref[...] = acc + sg * be_ref[0]


def _experts(disp, sgate, We, be):
    return pl.pallas_call(
        _expert_body,
        grid=(E + 1,),
        in_specs=[
            pl.BlockSpec((1, CAP, D), lambda i: (jnp.minimum(i, E - 1), 0, 0)),
            pl.BlockSpec((1, CAP, 1), lambda i: (jnp.minimum(i, E - 1), 0, 0)),
            pl.BlockSpec((1, D, D), lambda i: (jnp.minimum(i, E - 1), 0, 0)),
            pl.BlockSpec((1, 1, D), lambda i: (jnp.minimum(i, E - 1), 0, 0)),
        ],
        out_specs=pl.BlockSpec((CAP, D), lambda i: (i, 0)),
        out_shape=jax.ShapeDtypeStruct(((E + 1) * CAP, D), jnp.float32),
    )(disp, sgate, We, be)


def _combine_body(flat_hbm, eo_hbm, out_hbm, idx_v, buf0, buf1,
                  gsem0, gsem1, osem0, osem1):
    c = lax.axis_index("c")
    s = lax.axis_index("s")
    wid = s * 2 + c
    base = wid * TPW
    pltpu.sync_copy(flat_hbm.at[wid], idx_v)
    bufs = (buf0, buf1)
    gsems = (gsem0, gsem1)
    osems = (osem0, osem1)
    outcp = [None, None]
    incp = pltpu.async_copy(eo_hbm.at[idx_v.at[0]], buf0, gsem0)
    for j in range(NCH):
        b = j & 1
        nb = 1 - b
        incp.wait()
        if j + 1 < NCH:
            if outcp[nb] is not None:
                outcp[nb].wait()
            incp = pltpu.async_copy(
                eo_hbm.at[idx_v.at[j + 1]], bufs[nb], gsems[nb])
        outcp[b] = pltpu.async_copy(
            bufs[b], out_hbm.at[pl.ds(base + j * G, G)], osems[b])
    if outcp[(NCH - 2) & 1] is not None:
        outcp[(NCH - 2) & 1].wait()
    outcp[(NCH - 1) & 1].wait()


@functools.lru_cache(maxsize=1)
def _sc_kernels():
    mesh = plsc.VectorSubcoreMesh(core_axis_name="c", subcore_axis_name="s",
                                  num_cores=2)
    dispatch_k = pl.kernel(
        _dispatch_body,
        out_type=(jax.ShapeDtypeStruct((S,), jnp.float32),
                  jax.ShapeDtypeStruct((S, D), jnp.float32)),
        mesh=mesh,
        scratch_types=[
            pltpu.VMEM_SHARED((HS + T,), jnp.int32),
            pltpu.VMEM_SHARED((HS + T,), jnp.float32),
            pltpu.VMEM((NSC2, IDXW), jnp.int32),
            pltpu.VMEM((NSC2, IDXW), jnp.int32),
            pltpu.VMEM((NSC2, IDXW), jnp.float32),
            pltpu.VMEM((TPW,), jnp.int32),
            pltpu.VMEM((NCH, G), jnp.int32),
            pltpu.VMEM((TPW,), jnp.float32),
            pltpu.VMEM((G, D), jnp.float32),
            pltpu.VMEM((G, D), jnp.float32),
            pltpu.SemaphoreType.DMA,
            pltpu.SemaphoreType.DMA,
            pltpu.SemaphoreType.DMA,
            pltpu.SemaphoreType.DMA,
            pltpu.SemaphoreType.DMA,
        ],
    )
    combine_k = pl.kernel(
        _combine_body,
        out_type=jax.ShapeDtypeStruct((T, D), jnp.float32),
        mesh=mesh,
        scratch_types=[
            pltpu.VMEM((NCH, G), jnp.int32),
            pltpu.VMEM((G, D), jnp.float32),
            pltpu.VMEM((G, D), jnp.float32),
            pltpu.SemaphoreType.DMA,
            pltpu.SemaphoreType.DMA,
            pltpu.SemaphoreType.DMA,
            pltpu.SemaphoreType.DMA,
        ],
    )
    return dispatch_k, combine_k


def kernel(hidden_states, wg, We, be):
    B, SEQ, _ = hidden_states.shape
    x = hidden_states.reshape(T, D)
    dispatch_k, combine_k = _sc_kernels()

    scat, flat, gate, cnt, laux = _router(x, wg)

    scat_r = scat.reshape(16, NSC2, IDXW)
    gate_r = gate.reshape(16, NSC2, IDXW)
    slot_gate, disp = dispatch_k(scat_r, gate_r, x)

    disp3 = disp.reshape(E, CAP, D)
    sg3 = slot_gate.reshape(E, CAP, 1)
    eo = _experts(disp3, sg3, We, be.reshape(E, 1, D))

    flat_r = flat.reshape(NW, NCH, G)
    out = combine_k(flat_r, eo)

    return (out.reshape(B, SEQ, D), laux[0, 0], cnt.reshape(E))


# speedup vs baseline: 1.4400x; 1.0588x over previous
import functools

import jax
import jax.numpy as jnp
from jax import lax
from jax.experimental import pallas as pl
from jax.experimental.pallas import tpu as pltpu
from jax.experimental.pallas import tpu_sc as plsc

T = 16384
D = 768
E = 64
CAP = 256
S = E * CAP

BT = 1024
NB = T // BT

NW = 32
TPW = T // NW
G = 64
NCH = TPW // G
IDXW = 128
NSC = TPW // IDXW


def _router_body(x_ref, wg_ref, scat_ref, flat_ref, gate_ref, cnt_ref,
                 laux_ref, csum_ref, gsum_ref):
    b = pl.program_id(0)

    @pl.when(b == 0)
    def _init():
        csum_ref[...] = jnp.zeros_like(csum_ref)
        gsum_ref[...] = jnp.zeros_like(gsum_ref)

    logits = jnp.dot(x_ref[...], wg_ref[...],
                     preferred_element_type=jnp.float32)
    m = jnp.max(logits, axis=1, keepdims=True)
    p = jnp.exp(logits - m)
    gates = p / jnp.sum(p, axis=1, keepdims=True)

    gmax = jnp.max(gates, axis=1, keepdims=True)
    eidx = lax.broadcasted_iota(jnp.int32, (BT, E), 1)
    idx1 = jnp.min(jnp.where(gates >= gmax, eidx, E),
                   axis=1, keepdims=True)
    onehot = (eidx == idx1).astype(jnp.float32)

    cs = onehot
    k = 1
    while k < BT:
        cs = cs + jnp.pad(cs, ((k, 0), (0, 0)))[:BT]
        k *= 2

    base = csum_ref[...]
    loc = cs - 1.0 + base
    loc_tok = jnp.sum(loc * onehot, axis=1, keepdims=True)
    keep = loc_tok < float(CAP)
    loc_i = loc_tok.astype(jnp.int32)
    flat = idx1 * CAP + loc_i
    tok_id = lax.broadcasted_iota(jnp.int32, (BT, 1), 0) + b * BT

    scat_ref[...] = jnp.where(keep, flat, S + tok_id)
    flat_ref[...] = jnp.where(keep, flat, S)
    gate_ref[...] = jnp.where(keep, gmax, 0.0)

    csum_ref[...] = base + jnp.sum(onehot, axis=0, keepdims=True)
    gsum_ref[...] = gsum_ref[...] + jnp.sum(gates, axis=0, keepdims=True)

    @pl.when(b == NB - 1)
    def _fin():
        cnt = csum_ref[...]
        cnt_ref[...] = (cnt + 0.5).astype(jnp.int32)
        me = gsum_ref[...] * (1.0 / T)
        ce = cnt * (1.0 / T)
        laux_ref[...] = jnp.sum(me * ce, keepdims=True) * float(E)


def _router(x, wg):
    return pl.pallas_call(
        _router_body,
        grid=(NB,),
        in_specs=[
            pl.BlockSpec((BT, D), lambda i: (i, 0)),
            pl.BlockSpec((D, E), lambda i: (0, 0)),
        ],
        out_specs=[
            pl.BlockSpec((BT, 1), lambda i: (i, 0)),
            pl.BlockSpec((BT, 1), lambda i: (i, 0)),
            pl.BlockSpec((BT, 1), lambda i: (i, 0)),
            pl.BlockSpec((1, E), lambda i: (0, 0)),
            pl.BlockSpec((1, 1), lambda i: (0, 0)),
        ],
        out_shape=[
            jax.ShapeDtypeStruct((T, 1), jnp.int32),
            jax.ShapeDtypeStruct((T, 1), jnp.int32),
            jax.ShapeDtypeStruct((T, 1), jnp.float32),
            jax.ShapeDtypeStruct((1, E), jnp.int32),
            jax.ShapeDtypeStruct((1, 1), jnp.float32),
        ],
        scratch_shapes=[
            pltpu.VMEM((1, E), jnp.float32),
            pltpu.VMEM((1, E), jnp.float32),
        ],
    )(x, wg)


TPT = T // 16
NSC2 = TPT // IDXW
HS = S // 2


def _dispatch_body(scat_hbm, gate_hbm, x_hbm, sgate_hbm, disp_hbm,
                   map_sh, gate_sh,
                   idx_v, ids_v, g_v, gidx_v, gidx2_v, gstage_v, buf0, buf1,
                   ssem, gsem0, gsem1, osem0, osem1):
    c = lax.axis_index("c")
    s = lax.axis_index("s")

    tbase = s * TPT
    pltpu.sync_copy(scat_hbm.at[s], idx_v)
    pltpu.sync_copy(gate_hbm.at[s], g_v)
    lo = c * HS
    for j in range(NSC2):
        for i in range(IDXW // 16):
            off = pl.ds(i * 16, 16)
            tok = tbase + j * IDXW + i * 16 + lax.iota(jnp.int32, 16)
            v = idx_v[j, off]
            inhalf = jnp.logical_and(v >= lo, v < lo + HS)
            idx_v[j, off] = jnp.where(inhalf, v - lo, HS + tok)
            ids_v[j, off] = tok
    copies = []
    for j in range(NSC2):
        copies.append(pltpu.async_copy(ids_v.at[j], map_sh.at[idx_v.at[j]], ssem))
        copies.append(pltpu.async_copy(g_v.at[j], gate_sh.at[idx_v.at[j]], ssem))
    for cp in copies:
        cp.wait()
    plsc.subcore_barrier()

    lslot = s * TPW
    sbase = c * HS + lslot
    pltpu.sync_copy(map_sh.at[pl.ds(lslot, TPW)], gidx_v)
    pltpu.sync_copy(gate_sh.at[pl.ds(lslot, TPW)], gstage_v)
    pltpu.sync_copy(gstage_v, sgate_hbm.at[pl.ds(sbase, TPW)])

    for j in range(NCH):
        for i in range(G // 16):
            v = gidx_v[pl.ds(j * G + i * 16, 16)]
            gidx2_v[j, pl.ds(i * 16, 16)] = lax.bitwise_and(v, T - 1)

    bufs = (buf0, buf1)
    gsems = (gsem0, gsem1)
    osems = (osem0, osem1)
    outcp = [None, None]
    incp = pltpu.async_copy(x_hbm.at[gidx2_v.at[0]], buf0, gsem0)
    for j in range(NCH):
        b = j & 1
        nb = 1 - b
        incp.wait()
        if j + 1 < NCH:
            if outcp[nb] is not None:
                outcp[nb].wait()
            incp = pltpu.async_copy(
                x_hbm.at[gidx2_v.at[j + 1]], bufs[nb], gsems[nb])
        outcp[b] = pltpu.async_copy(
            bufs[b], disp_hbm.at[pl.ds(sbase + j * G, G)], osems[b])
    if outcp[(NCH - 2) & 1] is not None:
        outcp[(NCH - 2) & 1].wait()
    outcp[(NCH - 1) & 1].wait()


EPB = 2
NEB = E // EPB


def _expert_body(disp_ref, sg_ref, we_ref, be_ref, out_ref):
    e = pl.program_id(0)

    @pl.when(e == NEB)
    def _zero():
        out_ref[...] = jnp.zeros_like(out_ref)

    @pl.when(e < NEB)
    def _ffn():
        for k in range(EPB):
            xb = disp_ref[k]
            sg = sg_ref[k]
            acc = jnp.dot(xb * sg, we_ref[k],
                          preferred_element_type=jnp.float32)
            out_ref[pl.ds(k * CAP, CAP), :] = acc + sg * be_ref[k]


def _experts(disp, sgate, We, be):
    return pl.pallas_call(
        _expert_body,
        grid=(NEB + 1,),
        in_specs=[
            pl.BlockSpec((EPB, CAP, D), lambda i: (jnp.minimum(i, NEB - 1), 0, 0)),
            pl.BlockSpec((EPB, CAP, 1), lambda i: (jnp.minimum(i, NEB - 1), 0, 0)),
            pl.BlockSpec((EPB, D, D), lambda i: (jnp.minimum(i, NEB - 1), 0, 0)),
            pl.BlockSpec((EPB, 1, D), lambda i: (jnp.minimum(i, NEB - 1), 0, 0)),
        ],
        out_specs=pl.BlockSpec((EPB * CAP, D), lambda i: (i, 0)),
        out_shape=jax.ShapeDtypeStruct(((NEB + 1) * EPB * CAP, D), jnp.float32),
    )(disp, sgate, We, be)


def _combine_body(flat_hbm, eo_hbm, out_hbm, idx_v, buf0, buf1,
                  gsem0, gsem1, osem0, osem1):
    c = lax.axis_index("c")
    s = lax.axis_index("s")
    wid = s * 2 + c
    base = wid * TPW
    pltpu.sync_copy(flat_hbm.at[wid], idx_v)
    bufs = (buf0, buf1)
    gsems = (gsem0, gsem1)
    osems = (osem0, osem1)
    outcp = [None, None]
    incp = pltpu.async_copy(eo_hbm.at[idx_v.at[0]], buf0, gsem0)
    for j in range(NCH):
        b = j & 1
        nb = 1 - b
        incp.wait()
        if j + 1 < NCH:
            if outcp[nb] is not None:
                outcp[nb].wait()
            incp = pltpu.async_copy(
                eo_hbm.at[idx_v.at[j + 1]], bufs[nb], gsems[nb])
        outcp[b] = pltpu.async_copy(
            bufs[b], out_hbm.at[pl.ds(base + j * G, G)], osems[b])
    if outcp[(NCH - 2) & 1] is not None:
        outcp[(NCH - 2) & 1].wait()
    outcp[(NCH - 1) & 1].wait()


@functools.lru_cache(maxsize=1)
def _sc_kernels():
    mesh = plsc.VectorSubcoreMesh(core_axis_name="c", subcore_axis_name="s",
                                  num_cores=2)
    dispatch_k = pl.kernel(
        _dispatch_body,
        out_type=(jax.ShapeDtypeStruct((S,), jnp.float32),
                  jax.ShapeDtypeStruct((S, D), jnp.float32)),
        mesh=mesh,
        scratch_types=[
            pltpu.VMEM_SHARED((HS + T,), jnp.int32),
            pltpu.VMEM_SHARED((HS + T,), jnp.float32),
            pltpu.VMEM((NSC2, IDXW), jnp.int32),
            pltpu.VMEM((NSC2, IDXW), jnp.int32),
            pltpu.VMEM((NSC2, IDXW), jnp.float32),
            pltpu.VMEM((TPW,), jnp.int32),
            pltpu.VMEM((NCH, G), jnp.int32),
            pltpu.VMEM((TPW,), jnp.float32),
            pltpu.VMEM((G, D), jnp.float32),
            pltpu.VMEM((G, D), jnp.float32),
            pltpu.SemaphoreType.DMA,
            pltpu.SemaphoreType.DMA,
            pltpu.SemaphoreType.DMA,
            pltpu.SemaphoreType.DMA,
            pltpu.SemaphoreType.DMA,
        ],
    )
    combine_k = pl.kernel(
        _combine_body,
        out_type=jax.ShapeDtypeStruct((T, D), jnp.float32),
        mesh=mesh,
        scratch_types=[
            pltpu.VMEM((NCH, G), jnp.int32),
            pltpu.VMEM((G, D), jnp.float32),
            pltpu.VMEM((G, D), jnp.float32),
            pltpu.SemaphoreType.DMA,
            pltpu.SemaphoreType.DMA,
            pltpu.SemaphoreType.DMA,
            pltpu.SemaphoreType.DMA,
        ],
    )
    return dispatch_k, combine_k


def kernel(hidden_states, wg, We, be):
    B, SEQ, _ = hidden_states.shape
    x = hidden_states.reshape(T, D)
    dispatch_k, combine_k = _sc_kernels()

    scat, flat, gate, cnt, laux = _router(x, wg)

    scat_r = scat.reshape(16, NSC2, IDXW)
    gate_r = gate.reshape(16, NSC2, IDXW)
    slot_gate, disp = dispatch_k(scat_r, gate_r, x)

    disp3 = disp.reshape(E, CAP, D)
    sg3 = slot_gate.reshape(E, CAP, 1)
    eo = _experts(disp3, sg3, We, be.reshape(E, 1, D))

    flat_r = flat.reshape(NW, NCH, G)
    out = combine_k(flat_r, eo)

    return (out.reshape(B, SEQ, D), laux[0, 0], cnt.reshape(E))


# speedup vs baseline: 1.4513x; 1.0078x over previous
import functools

import jax
import jax.numpy as jnp
from jax import lax
from jax.experimental import pallas as pl
from jax.experimental.pallas import tpu as pltpu
from jax.experimental.pallas import tpu_sc as plsc

T = 16384
D = 768
E = 64
CAP = 256
S = E * CAP

BT = 1024
NB = T // BT

NW = 32
TPW = T // NW
G = 64
NCH = TPW // G
IDXW = 128
NSC = TPW // IDXW


def _router_body(x_ref, wg_ref, scat_ref, flat_ref, gate_ref, cnt_ref,
                 laux_ref, csum_ref, gsum_ref):
    b = pl.program_id(0)

    @pl.when(b == 0)
    def _init():
        csum_ref[...] = jnp.zeros_like(csum_ref)
        gsum_ref[...] = jnp.zeros_like(gsum_ref)

    logits = jnp.dot(x_ref[...], wg_ref[...],
                     preferred_element_type=jnp.float32)
    m = jnp.max(logits, axis=1, keepdims=True)
    p = jnp.exp(logits - m)
    gates = p / jnp.sum(p, axis=1, keepdims=True)

    gmax = jnp.max(gates, axis=1, keepdims=True)
    eidx = lax.broadcasted_iota(jnp.int32, (BT, E), 1)
    idx1 = jnp.min(jnp.where(gates >= gmax, eidx, E),
                   axis=1, keepdims=True)
    onehot = (eidx == idx1).astype(jnp.float32)

    cs = onehot
    k = 1
    while k < BT:
        cs = cs + jnp.pad(cs, ((k, 0), (0, 0)))[:BT]
        k *= 2

    base = csum_ref[...]
    loc = cs - 1.0 + base
    loc_tok = jnp.sum(loc * onehot, axis=1, keepdims=True)
    keep = loc_tok < float(CAP)
    loc_i = loc_tok.astype(jnp.int32)
    flat = idx1 * CAP + loc_i
    tok_id = lax.broadcasted_iota(jnp.int32, (BT, 1), 0) + b * BT

    scat_ref[...] = jnp.where(keep, flat, S + tok_id)
    flat_ref[...] = jnp.where(keep, flat, S)
    gate_ref[...] = jnp.where(keep, gmax, 0.0)

    csum_ref[...] = base + jnp.sum(onehot, axis=0, keepdims=True)
    gsum_ref[...] = gsum_ref[...] + jnp.sum(gates, axis=0, keepdims=True)

    @pl.when(b == NB - 1)
    def _fin():
        cnt = csum_ref[...]
        cnt_ref[...] = (cnt + 0.5).astype(jnp.int32)
        me = gsum_ref[...] * (1.0 / T)
        ce = cnt * (1.0 / T)
        laux_ref[...] = jnp.sum(me * ce, keepdims=True) * float(E)


def _router(x, wg):
    return pl.pallas_call(
        _router_body,
        grid=(NB,),
        in_specs=[
            pl.BlockSpec((BT, D), lambda i: (i, 0)),
            pl.BlockSpec((D, E), lambda i: (0, 0)),
        ],
        out_specs=[
            pl.BlockSpec((BT, 1), lambda i: (i, 0)),
            pl.BlockSpec((BT, 1), lambda i: (i, 0)),
            pl.BlockSpec((BT, 1), lambda i: (i, 0)),
            pl.BlockSpec((1, E), lambda i: (0, 0)),
            pl.BlockSpec((1, 1), lambda i: (0, 0)),
        ],
        out_shape=[
            jax.ShapeDtypeStruct((T, 1), jnp.int32),
            jax.ShapeDtypeStruct((T, 1), jnp.int32),
            jax.ShapeDtypeStruct((T, 1), jnp.float32),
            jax.ShapeDtypeStruct((1, E), jnp.int32),
            jax.ShapeDtypeStruct((1, 1), jnp.float32),
        ],
        scratch_shapes=[
            pltpu.VMEM((1, E), jnp.float32),
            pltpu.VMEM((1, E), jnp.float32),
        ],
    )(x, wg)


TPT = T // 16
NSC2 = TPT // IDXW
HS = S // 2


def _dispatch_body(scat_hbm, gate_hbm, x_hbm, sgate_hbm, disp_hbm,
                   map_sh, gate_sh,
                   idx_v, ids_v, g_v, gidx_v, gidx2_v, gstage_v, buf0, buf1,
                   ssem, gsem0, gsem1, osem0, osem1):
    c = lax.axis_index("c")
    s = lax.axis_index("s")

    tbase = s * TPT
    pltpu.sync_copy(scat_hbm.at[s], idx_v)
    pltpu.sync_copy(gate_hbm.at[s], g_v)
    lo = c * HS
    for j in range(NSC2):
        for i in range(IDXW // 16):
            off = pl.ds(i * 16, 16)
            tok = tbase + j * IDXW + i * 16 + lax.iota(jnp.int32, 16)
            v = idx_v[j, off]
            inhalf = jnp.logical_and(v >= lo, v < lo + HS)
            idx_v[j, off] = jnp.where(inhalf, v - lo, HS + tok)
            ids_v[j, off] = tok
    copies = []
    for j in range(NSC2):
        copies.append(pltpu.async_copy(ids_v.at[j], map_sh.at[idx_v.at[j]], ssem))
        copies.append(pltpu.async_copy(g_v.at[j], gate_sh.at[idx_v.at[j]], ssem))
    for cp in copies:
        cp.wait()
    plsc.subcore_barrier()

    lslot = s * TPW
    sbase = c * HS + lslot
    pltpu.sync_copy(map_sh.at[pl.ds(lslot, TPW)], gidx_v)
    pltpu.sync_copy(gate_sh.at[pl.ds(lslot, TPW)], gstage_v)
    pltpu.sync_copy(gstage_v, sgate_hbm.at[pl.ds(sbase, TPW)])

    for j in range(NCH):
        for i in range(G // 16):
            v = gidx_v[pl.ds(j * G + i * 16, 16)]
            gidx2_v[j, pl.ds(i * 16, 16)] = lax.bitwise_and(v, T - 1)

    bufs = (buf0, buf1)
    gsems = (gsem0, gsem1)
    osems = (osem0, osem1)
    outcp = [None, None]
    incp = pltpu.async_copy(x_hbm.at[gidx2_v.at[0]], buf0, gsem0)
    for j in range(NCH):
        b = j & 1
        nb = 1 - b
        incp.wait()
        if j + 1 < NCH:
            if outcp[nb] is not None:
                outcp[nb].wait()
            incp = pltpu.async_copy(
                x_hbm.at[gidx2_v.at[j + 1]], bufs[nb], gsems[nb])
        outcp[b] = pltpu.async_copy(
            bufs[b], disp_hbm.at[pl.ds(sbase + j * G, G)], osems[b])
    if outcp[(NCH - 2) & 1] is not None:
        outcp[(NCH - 2) & 1].wait()
    outcp[(NCH - 1) & 1].wait()


EPB = 4
NEB = E // EPB


def _expert_body(disp_ref, sg_ref, we_ref, be_ref, out_ref):
    e = pl.program_id(0)

    @pl.when(e == NEB)
    def _zero():
        out_ref[...] = jnp.zeros_like(out_ref)

    @pl.when(e < NEB)
    def _ffn():
        for k in range(EPB):
            xb = disp_ref[k]
            sg = sg_ref[k]
            acc = jnp.dot(xb * sg, we_ref[k],
                          preferred_element_type=jnp.float32)
            out_ref[pl.ds(k * CAP, CAP), :] = acc + sg * be_ref[k]


def _experts(disp, sgate, We, be):
    return pl.pallas_call(
        _expert_body,
        grid=(NEB + 1,),
        in_specs=[
            pl.BlockSpec((EPB, CAP, D), lambda i: (jnp.minimum(i, NEB - 1), 0, 0)),
            pl.BlockSpec((EPB, CAP, 1), lambda i: (jnp.minimum(i, NEB - 1), 0, 0)),
            pl.BlockSpec((EPB, D, D), lambda i: (jnp.minimum(i, NEB - 1), 0, 0)),
            pl.BlockSpec((EPB, 1, D), lambda i: (jnp.minimum(i, NEB - 1), 0, 0)),
        ],
        out_specs=pl.BlockSpec((EPB * CAP, D), lambda i: (i, 0)),
        out_shape=jax.ShapeDtypeStruct(((NEB + 1) * EPB * CAP, D), jnp.float32),
    )(disp, sgate, We, be)


def _combine_body(flat_hbm, eo_hbm, out_hbm, idx_v, buf0, buf1,
                  gsem0, gsem1, osem0, osem1):
    c = lax.axis_index("c")
    s = lax.axis_index("s")
    wid = s * 2 + c
    base = wid * TPW
    pltpu.sync_copy(flat_hbm.at[wid], idx_v)
    bufs = (buf0, buf1)
    gsems = (gsem0, gsem1)
    osems = (osem0, osem1)
    outcp = [None, None]
    incp = pltpu.async_copy(eo_hbm.at[idx_v.at[0]], buf0, gsem0)
    for j in range(NCH):
        b = j & 1
        nb = 1 - b
        incp.wait()
        if j + 1 < NCH:
            if outcp[nb] is not None:
                outcp[nb].wait()
            incp = pltpu.async_copy(
                eo_hbm.at[idx_v.at[j + 1]], bufs[nb], gsems[nb])
        outcp[b] = pltpu.async_copy(
            bufs[b], out_hbm.at[pl.ds(base + j * G, G)], osems[b])
    if outcp[(NCH - 2) & 1] is not None:
        outcp[(NCH - 2) & 1].wait()
    outcp[(NCH - 1) & 1].wait()


@functools.lru_cache(maxsize=1)
def _sc_kernels():
    mesh = plsc.VectorSubcoreMesh(core_axis_name="c", subcore_axis_name="s",
                                  num_cores=2)
    dispatch_k = pl.kernel(
        _dispatch_body,
        out_type=(jax.ShapeDtypeStruct((S,), jnp.float32),
                  jax.ShapeDtypeStruct((S, D), jnp.float32)),
        mesh=mesh,
        scratch_types=[
            pltpu.VMEM_SHARED((HS + T,), jnp.int32),
            pltpu.VMEM_SHARED((HS + T,), jnp.float32),
            pltpu.VMEM((NSC2, IDXW), jnp.int32),
            pltpu.VMEM((NSC2, IDXW), jnp.int32),
            pltpu.VMEM((NSC2, IDXW), jnp.float32),
            pltpu.VMEM((TPW,), jnp.int32),
            pltpu.VMEM((NCH, G), jnp.int32),
            pltpu.VMEM((TPW,), jnp.float32),
            pltpu.VMEM((G, D), jnp.float32),
            pltpu.VMEM((G, D), jnp.float32),
            pltpu.SemaphoreType.DMA,
            pltpu.SemaphoreType.DMA,
            pltpu.SemaphoreType.DMA,
            pltpu.SemaphoreType.DMA,
            pltpu.SemaphoreType.DMA,
        ],
    )
    combine_k = pl.kernel(
        _combine_body,
        out_type=jax.ShapeDtypeStruct((T, D), jnp.float32),
        mesh=mesh,
        scratch_types=[
            pltpu.VMEM((NCH, G), jnp.int32),
            pltpu.VMEM((G, D), jnp.float32),
            pltpu.VMEM((G, D), jnp.float32),
            pltpu.SemaphoreType.DMA,
            pltpu.SemaphoreType.DMA,
            pltpu.SemaphoreType.DMA,
            pltpu.SemaphoreType.DMA,
        ],
    )
    return dispatch_k, combine_k


def kernel(hidden_states, wg, We, be):
    B, SEQ, _ = hidden_states.shape
    x = hidden_states.reshape(T, D)
    dispatch_k, combine_k = _sc_kernels()

    scat, flat, gate, cnt, laux = _router(x, wg)

    scat_r = scat.reshape(16, NSC2, IDXW)
    gate_r = gate.reshape(16, NSC2, IDXW)
    slot_gate, disp = dispatch_k(scat_r, gate_r, x)

    disp3 = disp.reshape(E, CAP, D)
    sg3 = slot_gate.reshape(E, CAP, 1)
    eo = _experts(disp3, sg3, We, be.reshape(E, 1, D))

    flat_r = flat.reshape(NW, NCH, G)
    out = combine_k(flat_r, eo)

    return (out.reshape(B, SEQ, D), laux[0, 0], cnt.reshape(E))


# speedup vs baseline: 1.4756x; 1.0168x over previous
import functools

import jax
import jax.numpy as jnp
from jax import lax
from jax.experimental import pallas as pl
from jax.experimental.pallas import tpu as pltpu
from jax.experimental.pallas import tpu_sc as plsc

T = 16384
D = 768
E = 64
CAP = 256
S = E * CAP

BT = 2048
NB = T // BT

NW = 32
TPW = T // NW
G = 64
NCH = TPW // G
IDXW = 128
NSC = TPW // IDXW


def _router_body(x_ref, wg_ref, scat_ref, flat_ref, gate_ref, cnt_ref,
                 laux_ref, csum_ref, gsum_ref):
    b = pl.program_id(0)

    @pl.when(b == 0)
    def _init():
        csum_ref[...] = jnp.zeros_like(csum_ref)
        gsum_ref[...] = jnp.zeros_like(gsum_ref)

    logits = jnp.dot(x_ref[...], wg_ref[...],
                     preferred_element_type=jnp.float32)
    m = jnp.max(logits, axis=1, keepdims=True)
    p = jnp.exp(logits - m)
    gates = p / jnp.sum(p, axis=1, keepdims=True)

    gmax = jnp.max(gates, axis=1, keepdims=True)
    eidx = lax.broadcasted_iota(jnp.int32, (BT, E), 1)
    idx1 = jnp.min(jnp.where(gates >= gmax, eidx, E),
                   axis=1, keepdims=True)
    onehot = (eidx == idx1).astype(jnp.float32)

    cs = onehot
    k = 1
    while k < BT:
        cs = cs + jnp.pad(cs, ((k, 0), (0, 0)))[:BT]
        k *= 2

    base = csum_ref[...]
    loc = cs - 1.0 + base
    loc_tok = jnp.sum(loc * onehot, axis=1, keepdims=True)
    keep = loc_tok < float(CAP)
    loc_i = loc_tok.astype(jnp.int32)
    flat = idx1 * CAP + loc_i
    tok_id = lax.broadcasted_iota(jnp.int32, (BT, 1), 0) + b * BT

    scat_ref[...] = jnp.where(keep, flat, S + tok_id)
    flat_ref[...] = jnp.where(keep, flat, S)
    gate_ref[...] = jnp.where(keep, gmax, 0.0)

    csum_ref[...] = base + jnp.sum(onehot, axis=0, keepdims=True)
    gsum_ref[...] = gsum_ref[...] + jnp.sum(gates, axis=0, keepdims=True)

    @pl.when(b == NB - 1)
    def _fin():
        cnt = csum_ref[...]
        cnt_ref[...] = (cnt + 0.5).astype(jnp.int32)
        me = gsum_ref[...] * (1.0 / T)
        ce = cnt * (1.0 / T)
        laux_ref[...] = jnp.sum(me * ce, keepdims=True) * float(E)


def _router(x, wg):
    return pl.pallas_call(
        _router_body,
        grid=(NB,),
        in_specs=[
            pl.BlockSpec((BT, D), lambda i: (i, 0)),
            pl.BlockSpec((D, E), lambda i: (0, 0)),
        ],
        out_specs=[
            pl.BlockSpec((BT, 1), lambda i: (i, 0)),
            pl.BlockSpec((BT, 1), lambda i: (i, 0)),
            pl.BlockSpec((BT, 1), lambda i: (i, 0)),
            pl.BlockSpec((1, E), lambda i: (0, 0)),
            pl.BlockSpec((1, 1), lambda i: (0, 0)),
        ],
        out_shape=[
            jax.ShapeDtypeStruct((T, 1), jnp.int32),
            jax.ShapeDtypeStruct((T, 1), jnp.int32),
            jax.ShapeDtypeStruct((T, 1), jnp.float32),
            jax.ShapeDtypeStruct((1, E), jnp.int32),
            jax.ShapeDtypeStruct((1, 1), jnp.float32),
        ],
        scratch_shapes=[
            pltpu.VMEM((1, E), jnp.float32),
            pltpu.VMEM((1, E), jnp.float32),
        ],
    )(x, wg)


TPT = T // 16
NSC2 = TPT // IDXW
HS = S // 2


def _dispatch_body(scat_hbm, gate_hbm, x_hbm, sgate_hbm, disp_hbm,
                   map_sh, gate_sh,
                   idx_v, ids_v, g_v, gidx_v, gidx2_v, gstage_v, buf0, buf1,
                   ssem, gsem0, gsem1, osem0, osem1):
    c = lax.axis_index("c")
    s = lax.axis_index("s")

    tbase = s * TPT
    pltpu.sync_copy(scat_hbm.at[s], idx_v)
    pltpu.sync_copy(gate_hbm.at[s], g_v)
    lo = c * HS
    for j in range(NSC2):
        for i in range(IDXW // 16):
            off = pl.ds(i * 16, 16)
            tok = tbase + j * IDXW + i * 16 + lax.iota(jnp.int32, 16)
            v = idx_v[j, off]
            inhalf = jnp.logical_and(v >= lo, v < lo + HS)
            idx_v[j, off] = jnp.where(inhalf, v - lo, HS + tok)
            ids_v[j, off] = tok
    copies = []
    for j in range(NSC2):
        copies.append(pltpu.async_copy(ids_v.at[j], map_sh.at[idx_v.at[j]], ssem))
        copies.append(pltpu.async_copy(g_v.at[j], gate_sh.at[idx_v.at[j]], ssem))
    for cp in copies:
        cp.wait()
    plsc.subcore_barrier()

    lslot = s * TPW
    sbase = c * HS + lslot
    pltpu.sync_copy(map_sh.at[pl.ds(lslot, TPW)], gidx_v)
    pltpu.sync_copy(gate_sh.at[pl.ds(lslot, TPW)], gstage_v)
    pltpu.sync_copy(gstage_v, sgate_hbm.at[pl.ds(sbase, TPW)])

    for j in range(NCH):
        for i in range(G // 16):
            v = gidx_v[pl.ds(j * G + i * 16, 16)]
            gidx2_v[j, pl.ds(i * 16, 16)] = lax.bitwise_and(v, T - 1)

    bufs = (buf0, buf1)
    gsems = (gsem0, gsem1)
    osems = (osem0, osem1)
    outcp = [None, None]
    incp = pltpu.async_copy(x_hbm.at[gidx2_v.at[0]], buf0, gsem0)
    for j in range(NCH):
        b = j & 1
        nb = 1 - b
        incp.wait()
        if j + 1 < NCH:
            if outcp[nb] is not None:
                outcp[nb].wait()
            incp = pltpu.async_copy(
                x_hbm.at[gidx2_v.at[j + 1]], bufs[nb], gsems[nb])
        outcp[b] = pltpu.async_copy(
            bufs[b], disp_hbm.at[pl.ds(sbase + j * G, G)], osems[b])
    if outcp[(NCH - 2) & 1] is not None:
        outcp[(NCH - 2) & 1].wait()
    outcp[(NCH - 1) & 1].wait()


EPB = 4
NEB = E // EPB


def _expert_body(disp_ref, sg_ref, we_ref, be_ref, out_ref):
    e = pl.program_id(0)

    @pl.when(e == NEB)
    def _zero():
        out_ref[...] = jnp.zeros_like(out_ref)

    @pl.when(e < NEB)
    def _ffn():
        for k in range(EPB):
            xb = disp_ref[k]
            sg = sg_ref[k]
            acc = jnp.dot(xb * sg, we_ref[k],
                          preferred_element_type=jnp.float32)
            out_ref[pl.ds(k * CAP, CAP), :] = acc + sg * be_ref[k]


def _experts(disp, sgate, We, be):
    return pl.pallas_call(
        _expert_body,
        grid=(NEB + 1,),
        in_specs=[
            pl.BlockSpec((EPB, CAP, D), lambda i: (jnp.minimum(i, NEB - 1), 0, 0)),
            pl.BlockSpec((EPB, CAP, 1), lambda i: (jnp.minimum(i, NEB - 1), 0, 0)),
            pl.BlockSpec((EPB, D, D), lambda i: (jnp.minimum(i, NEB - 1), 0, 0)),
            pl.BlockSpec((EPB, 1, D), lambda i: (jnp.minimum(i, NEB - 1), 0, 0)),
        ],
        out_specs=pl.BlockSpec((EPB * CAP, D), lambda i: (i, 0)),
        out_shape=jax.ShapeDtypeStruct(((NEB + 1) * EPB * CAP, D), jnp.float32),
    )(disp, sgate, We, be)


def _combine_body(flat_hbm, eo_hbm, out_hbm, idx_v, buf0, buf1,
                  gsem0, gsem1, osem0, osem1):
    c = lax.axis_index("c")
    s = lax.axis_index("s")
    wid = s * 2 + c
    base = wid * TPW
    pltpu.sync_copy(flat_hbm.at[wid], idx_v)
    bufs = (buf0, buf1)
    gsems = (gsem0, gsem1)
    osems = (osem0, osem1)
    outcp = [None, None]
    incp = pltpu.async_copy(eo_hbm.at[idx_v.at[0]], buf0, gsem0)
    for j in range(NCH):
        b = j & 1
        nb = 1 - b
        incp.wait()
        if j + 1 < NCH:
            if outcp[nb] is not None:
                outcp[nb].wait()
            incp = pltpu.async_copy(
                eo_hbm.at[idx_v.at[j + 1]], bufs[nb], gsems[nb])
        outcp[b] = pltpu.async_copy(
            bufs[b], out_hbm.at[pl.ds(base + j * G, G)], osems[b])
    if outcp[(NCH - 2) & 1] is not None:
        outcp[(NCH - 2) & 1].wait()
    outcp[(NCH - 1) & 1].wait()


@functools.lru_cache(maxsize=1)
def _sc_kernels():
    mesh = plsc.VectorSubcoreMesh(core_axis_name="c", subcore_axis_name="s",
                                  num_cores=2)
    dispatch_k = pl.kernel(
        _dispatch_body,
        out_type=(jax.ShapeDtypeStruct((S,), jnp.float32),
                  jax.ShapeDtypeStruct((S, D), jnp.float32)),
        mesh=mesh,
        scratch_types=[
            pltpu.VMEM_SHARED((HS + T,), jnp.int32),
            pltpu.VMEM_SHARED((HS + T,), jnp.float32),
            pltpu.VMEM((NSC2, IDXW), jnp.int32),
            pltpu.VMEM((NSC2, IDXW), jnp.int32),
            pltpu.VMEM((NSC2, IDXW), jnp.float32),
            pltpu.VMEM((TPW,), jnp.int32),
            pltpu.VMEM((NCH, G), jnp.int32),
            pltpu.VMEM((TPW,), jnp.float32),
            pltpu.VMEM((G, D), jnp.float32),
            pltpu.VMEM((G, D), jnp.float32),
            pltpu.SemaphoreType.DMA,
            pltpu.SemaphoreType.DMA,
            pltpu.SemaphoreType.DMA,
            pltpu.SemaphoreType.DMA,
            pltpu.SemaphoreType.DMA,
        ],
    )
    combine_k = pl.kernel(
        _combine_body,
        out_type=jax.ShapeDtypeStruct((T, D), jnp.float32),
        mesh=mesh,
        scratch_types=[
            pltpu.VMEM((NCH, G), jnp.int32),
            pltpu.VMEM((G, D), jnp.float32),
            pltpu.VMEM((G, D), jnp.float32),
            pltpu.SemaphoreType.DMA,
            pltpu.SemaphoreType.DMA,
            pltpu.SemaphoreType.DMA,
            pltpu.SemaphoreType.DMA,
        ],
    )
    return dispatch_k, combine_k


def kernel(hidden_states, wg, We, be):
    B, SEQ, _ = hidden_states.shape
    x = hidden_states.reshape(T, D)
    dispatch_k, combine_k = _sc_kernels()

    scat, flat, gate, cnt, laux = _router(x, wg)

    scat_r = scat.reshape(16, NSC2, IDXW)
    gate_r = gate.reshape(16, NSC2, IDXW)
    slot_gate, disp = dispatch_k(scat_r, gate_r, x)

    disp3 = disp.reshape(E, CAP, D)
    sg3 = slot_gate.reshape(E, CAP, 1)
    eo = _experts(disp3, sg3, We, be.reshape(E, 1, D))

    flat_r = flat.reshape(NW, NCH, G)
    out = combine_k(flat_r, eo)

    return (out.reshape(B, SEQ, D), laux[0, 0], cnt.reshape(E))
